# Initial kernel scaffold; baseline (speedup 1.0000x reference)
import jax
import jax.numpy as jnp
from jax.experimental import pallas as pl


def kernel(x, analysis_window, synthesis_window):
    raise NotImplementedError("write your pallas kernel here")


# speedup vs baseline: 16.5709x; 16.5709x over previous
import functools

import jax
import jax.numpy as jnp
from jax import lax
from jax.experimental import pallas as pl
from jax.experimental.pallas import tpu as pltpu
from jax.experimental.pallas import tpu_sc as plsc

_HOP = 512
_L = 16


def _sc_body(x_hbm, a_hbm, s_hbm, o_hbm, xbuf, wall, abuf, sbuf, *,
             hop, per_worker, chunk, num_cores):
    wid = lax.axis_index("s") * num_cores + lax.axis_index("c")
    pltpu.sync_copy(a_hbm, abuf)
    pltpu.sync_copy(s_hbm, sbuf)
    for k in range(hop // _L):
        i = k * _L
        wlo = abuf[pl.ds(i, _L)] * sbuf[pl.ds(i, _L)]
        whi = abuf[pl.ds(hop + i, _L)] * sbuf[pl.ds(hop + i, _L)]
        wall[pl.ds(i, _L)] = wlo
        wall[pl.ds(hop + i, _L)] = wlo + whi
        wall[pl.ds(2 * hop + i, _L)] = whi

    periods_per_chunk = chunk // hop
    num_periods = per_worker // hop
    even = (wid % 2) == 0

    for c in range(per_worker // chunk):
        base = wid * per_worker + c * chunk
        pltpu.sync_copy(x_hbm.at[pl.ds(base, chunk)], xbuf)

        def period_body(p, _, c=c):
            g = c * periods_per_chunk + p
            off = jnp.where(
                even & (g == 0), 0,
                jnp.where(~even & (g == num_periods - 1), 2 * hop, hop))
            for k in range(hop // _L):
                i = p * hop + k * _L
                xbuf[pl.ds(i, _L)] = (
                    xbuf[pl.ds(i, _L)] * wall[pl.ds(off + k * _L, _L)])
            return 0

        lax.fori_loop(0, periods_per_chunk, period_body, 0)
        pltpu.sync_copy(xbuf, o_hbm.at[pl.ds(base, chunk)])


def kernel(x, analysis_window, synthesis_window):
    batch, num_samples = x.shape
    seg = analysis_window.shape[-1]
    hop = _HOP
    assert seg == 2 * hop and num_samples % hop == 0
    num_cores, num_subcores = 2, 16
    nw = num_cores * num_subcores
    total = batch * num_samples
    per_worker = total // nw
    assert num_samples % per_worker == 0
    chunk = 32768

    body = functools.partial(
        _sc_body, hop=hop, per_worker=per_worker, chunk=chunk,
        num_cores=num_cores)
    out = pl.kernel(
        body,
        mesh=plsc.VectorSubcoreMesh(
            core_axis_name="c", subcore_axis_name="s", num_cores=num_cores),
        out_type=jax.ShapeDtypeStruct((total,), x.dtype),
        scratch_types=[
            pltpu.VMEM((chunk,), jnp.float32),
            pltpu.VMEM((3 * hop,), jnp.float32),
            pltpu.VMEM((seg,), jnp.float32),
            pltpu.VMEM((seg,), jnp.float32),
        ],
    )(x.reshape(total), analysis_window, synthesis_window)
    return out.reshape(batch, num_samples)


# speedup vs baseline: 130.1830x; 7.8561x over previous
import functools

import jax
import jax.numpy as jnp
from jax import lax
from jax.experimental import pallas as pl

_HOP = 512


def _wola_block(x_ref, a_ref, s_ref, o_ref, *, hop, num_samples, block_cols):
    w = (a_ref[...] * s_ref[...]).reshape(2, hop)
    wlo = w[0:1, :]
    whi = w[1:2, :]
    wmid = wlo + whi
    reps = block_cols // hop
    tile = lambda v: jnp.broadcast_to(v.reshape(1, 1, hop), (1, reps, hop)).reshape(1, block_cols)
    col0 = pl.program_id(0) * block_cols
    t = col0 + lax.broadcasted_iota(jnp.int32, (1, block_cols), 1)
    weight = jnp.where(
        t < hop, tile(wlo), jnp.where(t >= num_samples - hop, tile(whi), tile(wmid))
    )
    o_ref[...] = x_ref[...] * weight


def kernel(x, analysis_window, synthesis_window):
    batch, num_samples = x.shape
    seg = analysis_window.shape[-1]
    hop = _HOP
    assert seg == 2 * hop and num_samples % hop == 0
    block_cols = 32768
    grid = num_samples // block_cols
    body = functools.partial(
        _wola_block, hop=hop, num_samples=num_samples, block_cols=block_cols
    )
    out = pl.pallas_call(
        body,
        grid=(grid,),
        in_specs=[
            pl.BlockSpec((batch, block_cols), lambda j: (0, j)),
            pl.BlockSpec((1, seg), lambda j: (0, 0)),
            pl.BlockSpec((1, seg), lambda j: (0, 0)),
        ],
        out_specs=pl.BlockSpec((batch, block_cols), lambda j: (0, j)),
        out_shape=jax.ShapeDtypeStruct((batch, num_samples), x.dtype),
    )(x, analysis_window.reshape(1, seg), synthesis_window.reshape(1, seg))
    return out
